# baseline (speedup 1.0000x reference)
import functools

import jax
import jax.numpy as jnp
from jax import lax
from jax.experimental import pallas as pl
from jax.experimental.pallas import tpu as pltpu
from jax.experimental.pallas import tpu_sc as plsc

B = 1024
K = 32
D = 512
JG = 40
JW = 48
JC = 33
NC = 2
NS = 16
NW = NC * NS
BPW = B // NW
L = 16


def _weights_body(nb_ref, nd_ref, w_ref):
    nb = nb_ref[...]
    nd = nd_ref[...]
    self_match = (nb == nd).astype(jnp.float32)
    cnt = self_match
    for k in range(K):
        cnt = cnt + (nb == nb[:, k:k + 1]).astype(jnp.float32)
    inv_nb = 1.0 / cnt
    cnt_self = 1.0 + jnp.sum(self_match, axis=1, keepdims=True)
    inv_self = 1.0 / cnt_self
    c = jnp.sum(inv_nb, axis=1, keepdims=True) + inv_self
    w_ref[...] = jnp.concatenate(
        [inv_nb / c, inv_self / c, jnp.zeros((B, JW - JC), jnp.float32)],
        axis=1,
    )


_weights = pl.pallas_call(
    _weights_body,
    out_shape=jax.ShapeDtypeStruct((B, JW), jnp.float32),
)


@functools.partial(
    pl.kernel,
    out_type=jax.ShapeDtypeStruct((B, D), jnp.float32),
    mesh=plsc.VectorSubcoreMesh(core_axis_name="c", subcore_axis_name="s"),
    scratch_types=[
        pltpu.VMEM((BPW, JG), jnp.int32),
        pltpu.VMEM((BPW * JW * L,), jnp.float32),
        pltpu.VMEM((JG, D), jnp.float32),
        pltpu.VMEM((BPW, D), jnp.float32),
        pltpu.SemaphoreType.DMA,
    ],
)
def _sc_aggregate(feat_hbm, ids_hbm, w_hbm, out_hbm,
                  ids_v, w_v, rows_v, obuf_v, gsem):
    wid = lax.axis_index("s") * NC + lax.axis_index("c")
    base = wid * BPW
    pltpu.sync_copy(ids_hbm.at[pl.ds(base, BPW)], ids_v)
    pltpu.sync_copy(w_hbm.at[pl.ds(base * JW * L, BPW * JW * L)], w_v)

    zero = jnp.zeros((L,), jnp.float32)

    def row_body(r, carry):
        pltpu.async_copy(feat_hbm.at[ids_v.at[r]], rows_v, gsem).wait()
        w_base = r * JW

        def zero_body(cc, c2):
            obuf_v[r, pl.ds(cc * L, L)] = zero
            return c2

        lax.fori_loop(0, D // L, zero_body, 0)

        def j_body(j, c2):
            wv = w_v[pl.ds((w_base + j) * L, L)]

            def cc_body(cc, c3):
                plsc.addupdate(
                    obuf_v.at[r, pl.ds(cc * L, L)],
                    wv * rows_v[j, pl.ds(cc * L, L)])
                return c3

            lax.fori_loop(0, D // L, cc_body, 0)
            return c2

        lax.fori_loop(0, JC, j_body, 0)
        return carry

    lax.fori_loop(0, BPW, row_body, 0)
    pltpu.sync_copy(obuf_v, out_hbm.at[pl.ds(base, BPW)])


def kernel(raw_features, nodes, neighbors):
    nb = neighbors.astype(jnp.int32)
    nd = nodes.astype(jnp.int32).reshape(B, 1)
    ids = jnp.concatenate(
        [nb, nd, jnp.zeros((B, JG - JC), jnp.int32)], axis=1)
    w = _weights(nb, nd)
    w_exp = jnp.broadcast_to(w[:, :, None], (B, JW, L)).reshape(B * JW * L)
    return _sc_aggregate(raw_features, ids, w_exp)


# speedup vs baseline: 1.0023x; 1.0023x over previous
import functools

import jax
import jax.numpy as jnp
from jax import lax
from jax.experimental import pallas as pl
from jax.experimental.pallas import tpu as pltpu
from jax.experimental.pallas import tpu_sc as plsc

B = 1024
K = 32
D = 512
JG = 40
JC = 33
NC = 2
NS = 16
NW = NC * NS
BPW = B // NW
NP = BPW // 2
L = 16


def _weights_body(nb_ref, nd_ref, w_ref):
    nb = nb_ref[...]
    nd = nd_ref[...]
    self_match = (nb == nd).astype(jnp.float32)
    cnt = self_match
    for k in range(K):
        cnt = cnt + (nb == nb[:, k:k + 1]).astype(jnp.float32)
    inv_nb = 1.0 / cnt
    cnt_self = 1.0 + jnp.sum(self_match, axis=1, keepdims=True)
    inv_self = 1.0 / cnt_self
    c = jnp.sum(inv_nb, axis=1, keepdims=True) + inv_self
    w_ref[...] = jnp.concatenate([inv_nb / c, inv_self / c], axis=1)


_weights = pl.pallas_call(
    _weights_body,
    out_shape=jax.ShapeDtypeStruct((B, JC), jnp.float32),
)


@functools.partial(
    pl.kernel,
    out_type=jax.ShapeDtypeStruct((B, D), jnp.float32),
    mesh=plsc.VectorSubcoreMesh(core_axis_name="c", subcore_axis_name="s"),
    scratch_types=[
        pltpu.VMEM((BPW, JG), jnp.int32),
        pltpu.VMEM((BPW * JC * L,), jnp.float32),
        pltpu.VMEM((2, JG, D), jnp.float32),
        pltpu.VMEM((BPW, D), jnp.float32),
        pltpu.SemaphoreType.DMA,
        pltpu.SemaphoreType.DMA,
    ],
)
def _sc_aggregate(feat_hbm, ids_hbm, w_hbm, out_hbm,
                  ids_v, w_v, rows_v, obuf_v, sem0, sem1):
    wid = lax.axis_index("s") * NC + lax.axis_index("c")
    base = wid * BPW
    pltpu.sync_copy(ids_hbm.at[pl.ds(base, BPW)], ids_v)
    pltpu.sync_copy(w_hbm.at[pl.ds(base * JC * L, BPW * JC * L)], w_v)

    def gather(r, buf, sem):
        pltpu.async_copy(
            feat_hbm.at[ids_v.at[r]], rows_v.at[buf], sem)

    def gather_wait(buf, sem):
        pltpu.make_async_copy(
            feat_hbm.at[ids_v.at[0]], rows_v.at[buf], sem
        ).wait()

    def compute(r, buf):
        wvs = [w_v[pl.ds((r * JC + j) * L, L)] for j in range(JC)]

        def cc_body(cc, c3):
            off = cc * L
            acc = wvs[0] * rows_v[buf, 0, pl.ds(off, L)]
            for j in range(1, JC):
                acc = acc + wvs[j] * rows_v[buf, j, pl.ds(off, L)]
            obuf_v[r, pl.ds(off, L)] = acc
            return c3

        lax.fori_loop(0, D // L, cc_body, 0)

    gather(0, 0, sem0)

    def pair_body(p, carry):
        r0 = 2 * p
        gather(r0 + 1, 1, sem1)
        gather_wait(0, sem0)
        compute(r0, 0)

        @pl.when(p < NP - 1)
        def _():
            gather(r0 + 2, 0, sem0)

        gather_wait(1, sem1)
        compute(r0 + 1, 1)
        return carry

    lax.fori_loop(0, NP, pair_body, 0)
    pltpu.sync_copy(obuf_v, out_hbm.at[pl.ds(base, BPW)])


def kernel(raw_features, nodes, neighbors):
    nb = neighbors.astype(jnp.int32)
    nd = nodes.astype(jnp.int32).reshape(B, 1)
    ids = jnp.concatenate(
        [nb, nd, jnp.zeros((B, JG - JC), jnp.int32)], axis=1)
    w = _weights(nb, nd)
    w_exp = jnp.broadcast_to(w[:, :, None], (B, JC, L)).reshape(B * JC * L)
    return _sc_aggregate(raw_features, ids, w_exp)


# speedup vs baseline: 1.0122x; 1.0099x over previous
import functools

import jax
import jax.numpy as jnp
from jax import lax
from jax.experimental import pallas as pl
from jax.experimental.pallas import tpu as pltpu
from jax.experimental.pallas import tpu_sc as plsc

B = 1024
K = 32
D = 512
JG = 40
JC = 33
NC = 2
NS = 16
NW = NC * NS
BPW = B // NW
NP = BPW // 2
L = 16


def _weights_body(nb_ref, nd_ref, w_ref):
    nb = nb_ref[...]
    nd = nd_ref[...]
    self_match = (nb == nd).astype(jnp.float32)
    cnt = self_match
    for k in range(K):
        cnt = cnt + (nb == nb[:, k:k + 1]).astype(jnp.float32)
    inv_nb = 1.0 / cnt
    cnt_self = 1.0 + jnp.sum(self_match, axis=1, keepdims=True)
    inv_self = 1.0 / cnt_self
    c = jnp.sum(inv_nb, axis=1, keepdims=True) + inv_self
    w_ref[...] = jnp.concatenate([inv_nb / c, inv_self / c], axis=1)


_weights = pl.pallas_call(
    _weights_body,
    out_shape=jax.ShapeDtypeStruct((B, JC), jnp.float32),
)


@functools.partial(
    pl.kernel,
    out_type=jax.ShapeDtypeStruct((B, D), jnp.float32),
    mesh=plsc.VectorSubcoreMesh(core_axis_name="c", subcore_axis_name="s"),
    scratch_types=[
        pltpu.VMEM((BPW * JG,), jnp.int32),
        pltpu.VMEM((BPW * JC * L,), jnp.float32),
        pltpu.VMEM((2, 2 * JG, D), jnp.float32),
        pltpu.VMEM((BPW, D), jnp.float32),
        pltpu.SemaphoreType.DMA,
        pltpu.SemaphoreType.DMA,
    ],
)
def _sc_aggregate(feat_hbm, ids_hbm, w_hbm, out_hbm,
                  ids_v, w_v, rows_v, obuf_v, sem0, sem1):
    wid = lax.axis_index("s") * NC + lax.axis_index("c")
    base = wid * BPW
    pltpu.sync_copy(ids_hbm.at[pl.ds(base * JG, BPW * JG)], ids_v)
    pltpu.sync_copy(w_hbm.at[pl.ds(base * JC * L, BPW * JC * L)], w_v)

    def gather(p, buf, sem):
        pltpu.async_copy(
            feat_hbm.at[ids_v.at[pl.ds(p * 2 * JG, 2 * JG)]],
            rows_v.at[buf], sem)

    def gather_wait(buf, sem):
        pltpu.make_async_copy(
            feat_hbm.at[ids_v.at[pl.ds(0, 2 * JG)]], rows_v.at[buf], sem
        ).wait()

    def compute(r, buf, half):
        wvs = [w_v[pl.ds((r * JC + j) * L, L)] for j in range(JC)]

        def cc_body(cc, c3):
            off = cc * L
            acc = wvs[0] * rows_v[buf, half * JG, pl.ds(off, L)]
            for j in range(1, JC):
                acc = acc + wvs[j] * rows_v[buf, half * JG + j, pl.ds(off, L)]
            obuf_v[r, pl.ds(off, L)] = acc
            return c3

        lax.fori_loop(0, D // L, cc_body, 0)

    gather(0, 0, sem0)

    def quad_body(q, carry):
        p0 = 2 * q
        p1 = p0 + 1
        gather(p1, 1, sem1)
        gather_wait(0, sem0)
        compute(2 * p0, 0, 0)
        compute(2 * p0 + 1, 0, 1)

        @pl.when(q < NP // 2 - 1)
        def _():
            gather(p0 + 2, 0, sem0)

        gather_wait(1, sem1)
        compute(2 * p1, 1, 0)
        compute(2 * p1 + 1, 1, 1)
        return carry

    lax.fori_loop(0, NP // 2, quad_body, 0)
    pltpu.sync_copy(obuf_v, out_hbm.at[pl.ds(base, BPW)])


def kernel(raw_features, nodes, neighbors):
    nb = neighbors.astype(jnp.int32)
    nd = nodes.astype(jnp.int32).reshape(B, 1)
    ids = jnp.concatenate(
        [nb, nd, jnp.zeros((B, JG - JC), jnp.int32)], axis=1).reshape(B * JG)
    w = _weights(nb, nd)
    w_exp = jnp.broadcast_to(w[:, :, None], (B, JC, L)).reshape(B * JC * L)
    return _sc_aggregate(raw_features, ids, w_exp)


# speedup vs baseline: 3.6548x; 3.6109x over previous
import functools

import jax
import jax.numpy as jnp
from jax import lax
from jax.experimental import pallas as pl
from jax.experimental.pallas import tpu as pltpu
from jax.experimental.pallas import tpu_sc as plsc

B = 1024
N_FEAT_ROWS = 10000
K = 32
D = 512
JG = 40
JC = 33
NC = 2
NS = 16
NW = NC * NS
BPW = B // NW
NP = BPW // 2
L = 16


def _weights_body(nb_ref, nd_ref, w_ref):
    nb = nb_ref[...]
    nd = nd_ref[...]
    self_match = (nb == nd).astype(jnp.float32)
    cnt = self_match
    for k in range(K):
        cnt = cnt + (nb == nb[:, k:k + 1]).astype(jnp.float32)
    inv_nb = 1.0 / cnt
    cnt_self = 1.0 + jnp.sum(self_match, axis=1, keepdims=True)
    inv_self = 1.0 / cnt_self
    c = jnp.sum(inv_nb, axis=1, keepdims=True) + inv_self
    w_ref[...] = jnp.concatenate([inv_nb / c, inv_self / c], axis=1)


_weights = pl.pallas_call(
    _weights_body,
    out_shape=jax.ShapeDtypeStruct((B, JC), jnp.float32),
)


@functools.partial(
    pl.kernel,
    out_type=jax.ShapeDtypeStruct((B, D), jnp.float32),
    mesh=plsc.VectorSubcoreMesh(core_axis_name="c", subcore_axis_name="s"),
    scratch_types=[
        pltpu.VMEM((BPW * JG,), jnp.int32),
        pltpu.VMEM((BPW * JC * L,), jnp.float32),
        pltpu.VMEM((2, 2 * JG, D), jnp.float32),
        pltpu.VMEM((BPW, D), jnp.float32),
        pltpu.SemaphoreType.DMA,
        pltpu.SemaphoreType.DMA,
    ],
)
def _sc_aggregate(feat_hbm, ids_hbm, w_hbm, out_hbm,
                  ids_v, w_v, rows_v, obuf_v, sem0, sem1):
    wid = lax.axis_index("s") * NC + lax.axis_index("c")
    base = wid * BPW
    pltpu.sync_copy(ids_hbm.at[pl.ds(base * JG, BPW * JG)], ids_v)
    pltpu.sync_copy(w_hbm.at[pl.ds(base * JC * L, BPW * JC * L)], w_v)

    def gather(p, buf, sem):
        pltpu.async_copy(
            feat_hbm.at[ids_v.at[pl.ds(p * 2 * JG, 2 * JG)]],
            rows_v.at[buf], sem)

    def gather_wait(buf, sem):
        pltpu.make_async_copy(
            feat_hbm.at[ids_v.at[pl.ds(0, 2 * JG)]], rows_v.at[buf], sem
        ).wait()

    def compute(r, buf, half):
        wvs = [w_v[pl.ds((r * JC + j) * L, L)] for j in range(JC)]

        def cc_body(cc, c3):
            off = cc * L
            acc = wvs[0] * rows_v[buf, half * JG, pl.ds(off, L)]
            for j in range(1, JC):
                acc = acc + wvs[j] * rows_v[buf, half * JG + j, pl.ds(off, L)]
            obuf_v[r, pl.ds(off, L)] = acc
            return c3

        lax.fori_loop(0, D // L, cc_body, 0)

    gather(0, 0, sem0)

    def quad_body(q, carry):
        p0 = 2 * q
        p1 = p0 + 1
        gather(p1, 1, sem1)
        gather_wait(0, sem0)
        compute(2 * p0, 0, 0)
        compute(2 * p0 + 1, 0, 1)

        @pl.when(q < NP // 2 - 1)
        def _():
            gather(p0 + 2, 0, sem0)

        gather_wait(1, sem1)
        compute(2 * p1, 1, 0)
        compute(2 * p1 + 1, 1, 1)
        return carry

    lax.fori_loop(0, NP // 2, quad_body, 0)
    pltpu.sync_copy(obuf_v, out_hbm.at[pl.ds(base, BPW)])


def kernel(raw_features, nodes, neighbors):
    nb = neighbors.astype(jnp.int32)
    nd = nodes.astype(jnp.int32).reshape(B, 1)
    pads = (jnp.arange(B, dtype=jnp.int32)[:, None] * (JG - JC)
            + jnp.arange(JG - JC, dtype=jnp.int32)[None, :]) % N_FEAT_ROWS
    ids = jnp.concatenate([nb, nd, pads], axis=1).reshape(B * JG)
    w = _weights(nb, nd)
    w_exp = jnp.broadcast_to(w[:, :, None], (B, JC, L)).reshape(B * JC * L)
    return _sc_aggregate(raw_features, ids, w_exp)
